# baseline (device time: 158032 ns/iter reference)
import jax
import jax.numpy as jnp
from jax import lax
from jax.experimental import pallas as pl
from jax.experimental.pallas import tpu as pltpu

N_DEV = 4


def kernel(t, W):
    m_per, k = t.shape
    _, n = W.shape
    mc = m_per // N_DEV

    def body(t_ref, w_ref, out_ref, rs_buf, rs_send, rs_recv, ag_send, ag_recv):
        my = lax.axis_index("i")
        left = lax.rem(my + N_DEV - 1, N_DEV)
        right = lax.rem(my + 1, N_DEV)

        barrier_sem = pltpu.get_barrier_semaphore()
        for nbr in (left, right):
            pl.semaphore_signal(
                barrier_sem, inc=1,
                device_id=(nbr,), device_id_type=pl.DeviceIdType.MESH,
            )
        pl.semaphore_wait(barrier_sem, 2)

        for s in range(N_DEV - 1):
            send_chunk = lax.rem(my - s + N_DEV, N_DEV)
            if s == 0:
                src = t_ref.at[pl.ds(send_chunk * mc, mc), :]
            else:
                src = rs_buf.at[s - 1]
            rdma = pltpu.make_async_remote_copy(
                src_ref=src,
                dst_ref=rs_buf.at[s],
                send_sem=rs_send.at[s],
                recv_sem=rs_recv.at[s],
                device_id=(right,),
                device_id_type=pl.DeviceIdType.MESH,
            )
            rdma.start()
            rdma.wait()
            recv_chunk = lax.rem(my - s - 1 + N_DEV, N_DEV)
            rs_buf[s, :, :] = rs_buf[s, :, :] + t_ref[pl.ds(recv_chunk * mc, mc), :]

        my_chunk = lax.rem(my + 1, N_DEV)
        y = lax.dot_general(
            rs_buf[N_DEV - 2, :, :], w_ref[:, :],
            dimension_numbers=(((1,), (0,)), ((), ())),
            preferred_element_type=jnp.float32,
        )
        out_ref[pl.ds(my_chunk * mc, mc), :] = y

        for h in range(N_DEV - 1):
            c = lax.rem(my + 1 - h + N_DEV, N_DEV)
            rdma = pltpu.make_async_remote_copy(
                src_ref=out_ref.at[pl.ds(c * mc, mc), :],
                dst_ref=out_ref.at[pl.ds(c * mc, mc), :],
                send_sem=ag_send.at[h],
                recv_sem=ag_recv.at[h],
                device_id=(right,),
                device_id_type=pl.DeviceIdType.MESH,
            )
            rdma.start()
            rdma.wait()

    return pl.pallas_call(
        body,
        out_shape=jax.ShapeDtypeStruct((m_per, n), jnp.float32),
        in_specs=[
            pl.BlockSpec(memory_space=pltpu.VMEM),
            pl.BlockSpec(memory_space=pltpu.VMEM),
        ],
        out_specs=pl.BlockSpec(memory_space=pltpu.VMEM),
        scratch_shapes=[
            pltpu.VMEM((N_DEV - 1, mc, k), jnp.float32),
            pltpu.SemaphoreType.DMA((N_DEV - 1,)),
            pltpu.SemaphoreType.DMA((N_DEV - 1,)),
            pltpu.SemaphoreType.DMA((N_DEV - 1,)),
            pltpu.SemaphoreType.DMA((N_DEV - 1,)),
        ],
        compiler_params=pltpu.CompilerParams(collective_id=0),
    )(t, W)


# device time: 87938 ns/iter; 1.7971x vs baseline; 1.7971x over previous
import jax
import jax.numpy as jnp
from jax import lax
from jax.experimental import pallas as pl
from jax.experimental.pallas import tpu as pltpu

N_DEV = 4


def kernel(t, W):
    m_per, k = t.shape
    _, n = W.shape
    mh = m_per // 2
    mq = m_per // 4
    me = m_per // 8

    def body(t_ref, w_ref, out_ref, rs1, rs2, sems_s, sems_r):
        my = lax.axis_index("i")
        p_a = my ^ 1
        p_b = 3 - my
        a_bit = my & 1
        b_bit = my // 2
        keep1 = a_bit ^ b_bit
        keep2 = b_bit

        barrier_sem = pltpu.get_barrier_semaphore()
        for nbr in (p_a, p_b):
            pl.semaphore_signal(
                barrier_sem, inc=1,
                device_id=(nbr,), device_id_type=pl.DeviceIdType.MESH,
            )
        pl.semaphore_wait(barrier_sem, 2)

        def xchg(sem_idx, src, dst, target):
            return pltpu.make_async_remote_copy(
                src_ref=src, dst_ref=dst,
                send_sem=sems_s.at[sem_idx], recv_sem=sems_r.at[sem_idx],
                device_id=(target,), device_id_type=pl.DeviceIdType.MESH,
            )

        r1 = xchg(0, t_ref.at[pl.ds((1 - keep1) * mq, mq), :], rs1.at[0], p_a)
        r2 = xchg(1, t_ref.at[pl.ds(mh + (1 - keep2) * mq, mq), :], rs1.at[1], p_b)
        r1.start()
        r2.start()
        r1.wait()
        r2.wait()
        rs1[0, :, :] = rs1[0, :, :] + t_ref[pl.ds(keep1 * mq, mq), :]
        rs1[1, :, :] = rs1[1, :, :] + t_ref[pl.ds(mh + keep2 * mq, mq), :]

        r3 = xchg(2, rs1.at[0, pl.ds((1 - b_bit) * me, me), :], rs2.at[0], p_b)
        r4 = xchg(3, rs1.at[1, pl.ds((1 - a_bit) * me, me), :], rs2.at[1], p_a)
        r3.start()
        r4.start()
        r3.wait()
        r4.wait()

        row1 = keep1 * mq + b_bit * me
        row2 = mh + keep2 * mq + a_bit * me

        s1 = rs2[0, :, :] + rs1[0, pl.ds(b_bit * me, me), :]
        y1 = lax.dot_general(
            s1, w_ref[:, :],
            dimension_numbers=(((1,), (0,)), ((), ())),
            preferred_element_type=jnp.float32,
        )
        out_ref[pl.ds(row1, me), :] = y1
        g1 = xchg(4, out_ref.at[pl.ds(row1, me), :],
                  out_ref.at[pl.ds(row1, me), :], p_b)
        g1.start()

        s2 = rs2[1, :, :] + rs1[1, pl.ds(a_bit * me, me), :]
        y2 = lax.dot_general(
            s2, w_ref[:, :],
            dimension_numbers=(((1,), (0,)), ((), ())),
            preferred_element_type=jnp.float32,
        )
        out_ref[pl.ds(row2, me), :] = y2
        g2 = xchg(5, out_ref.at[pl.ds(row2, me), :],
                  out_ref.at[pl.ds(row2, me), :], p_a)
        g2.start()

        g1.wait()
        g2.wait()

        run1 = keep1 * mq
        run2 = mh + keep2 * mq
        g3 = xchg(6, out_ref.at[pl.ds(run1, mq), :],
                  out_ref.at[pl.ds(run1, mq), :], p_a)
        g4 = xchg(7, out_ref.at[pl.ds(run2, mq), :],
                  out_ref.at[pl.ds(run2, mq), :], p_b)
        g3.start()
        g4.start()
        g3.wait()
        g4.wait()

    return pl.pallas_call(
        body,
        out_shape=jax.ShapeDtypeStruct((m_per, n), jnp.float32),
        in_specs=[
            pl.BlockSpec(memory_space=pltpu.VMEM),
            pl.BlockSpec(memory_space=pltpu.VMEM),
        ],
        out_specs=pl.BlockSpec(memory_space=pltpu.VMEM),
        scratch_shapes=[
            pltpu.VMEM((2, m_per // 4, k), jnp.float32),
            pltpu.VMEM((2, m_per // 8, k), jnp.float32),
            pltpu.SemaphoreType.DMA((8,)),
            pltpu.SemaphoreType.DMA((8,)),
        ],
        compiler_params=pltpu.CompilerParams(collective_id=0),
    )(t, W)


# device time: 54099 ns/iter; 2.9212x vs baseline; 1.6255x over previous
import jax
import jax.numpy as jnp
from jax import lax
from jax.experimental import pallas as pl
from jax.experimental.pallas import tpu as pltpu

N_DEV = 4


def kernel(t, W):
    m_per, k = t.shape
    _, n = W.shape
    mh = m_per // 2
    mq = m_per // 4
    me = m_per // 8

    def body(t_ref, w_ref, out_ref, c1s, rs1r, rs2s, rs2r, yb, sems_s, sems_r):
        my = lax.axis_index("i")
        p_a = my ^ 1
        p_b = 3 - my
        a_bit = my & 1
        b_bit = my // 2
        keep1 = a_bit ^ b_bit
        keep2 = b_bit
        q1 = b_bit
        q2 = a_bit
        row1 = keep1 * mq + q1 * me
        row2 = mh + keep2 * mq + q2 * me
        l2 = keep2 * mq + q2 * me

        barrier_sem = pltpu.get_barrier_semaphore()
        for nbr in (p_a, p_b):
            pl.semaphore_signal(
                barrier_sem, inc=1,
                device_id=(nbr,), device_id_type=pl.DeviceIdType.MESH,
            )
        pl.semaphore_wait(barrier_sem, 2)

        def xchg(sem_idx, src, dst, target):
            return pltpu.make_async_remote_copy(
                src_ref=src, dst_ref=dst,
                send_sem=sems_s.at[sem_idx], recv_sem=sems_r.at[sem_idx],
                device_id=(target,), device_id_type=pl.DeviceIdType.MESH,
            )

        c1s[0, :, :] = t_ref[pl.ds((1 - keep1) * mq, mq), :].astype(jnp.bfloat16)
        r1 = xchg(0, c1s.at[0], rs1r.at[0], p_a)
        r1.start()
        c1s[1, :, :] = t_ref[pl.ds(mh + (1 - keep2) * mq, mq), :].astype(
            jnp.bfloat16
        )
        r2 = xchg(1, c1s.at[1], rs1r.at[1], p_b)
        r2.start()

        r1.wait()
        rs2s[0, :, :] = (
            rs1r[0, pl.ds((1 - q1) * me, me), :].astype(jnp.float32)
            + t_ref[pl.ds(keep1 * mq + (1 - q1) * me, me), :]
        ).astype(jnp.bfloat16)
        r3 = xchg(2, rs2s.at[0], rs2r.at[0], p_b)
        r3.start()

        r2.wait()
        rs2s[1, :, :] = (
            rs1r[1, pl.ds((1 - q2) * me, me), :].astype(jnp.float32)
            + t_ref[pl.ds(mh + keep2 * mq + (1 - q2) * me, me), :]
        ).astype(jnp.bfloat16)
        r4 = xchg(3, rs2s.at[1], rs2r.at[1], p_a)
        r4.start()

        r3.wait()
        s1 = (
            rs2r[0, :, :].astype(jnp.float32)
            + rs1r[0, pl.ds(q1 * me, me), :].astype(jnp.float32)
            + t_ref[pl.ds(row1, me), :]
        )
        y1 = lax.dot_general(
            s1, w_ref[:, :],
            dimension_numbers=(((1,), (0,)), ((), ())),
            preferred_element_type=jnp.float32,
        )
        out_ref[pl.ds(row1, me), :] = y1
        yb[0, pl.ds(row1, me), :] = y1.astype(jnp.bfloat16)
        g1 = xchg(4, yb.at[0, pl.ds(row1, me), :],
                  yb.at[0, pl.ds(row1, me), :], p_b)
        g1.start()

        r4.wait()
        s2 = (
            rs2r[1, :, :].astype(jnp.float32)
            + rs1r[1, pl.ds(q2 * me, me), :].astype(jnp.float32)
            + t_ref[pl.ds(row2, me), :]
        )
        y2 = lax.dot_general(
            s2, w_ref[:, :],
            dimension_numbers=(((1,), (0,)), ((), ())),
            preferred_element_type=jnp.float32,
        )
        out_ref[pl.ds(row2, me), :] = y2
        yb[1, pl.ds(l2, me), :] = y2.astype(jnp.bfloat16)
        g2 = xchg(5, yb.at[1, pl.ds(l2, me), :],
                  yb.at[1, pl.ds(l2, me), :], p_a)
        g2.start()

        g1.wait()
        g3 = xchg(6, yb.at[0, pl.ds(keep1 * mq, mq), :],
                  yb.at[0, pl.ds(keep1 * mq, mq), :], p_a)
        g3.start()
        pq1 = keep1 * mq + (1 - q1) * me
        out_ref[pl.ds(pq1, me), :] = yb[0, pl.ds(pq1, me), :].astype(jnp.float32)

        g2.wait()
        g4 = xchg(7, yb.at[1, pl.ds(keep2 * mq, mq), :],
                  yb.at[1, pl.ds(keep2 * mq, mq), :], p_b)
        g4.start()
        pq2 = keep2 * mq + (1 - q2) * me
        out_ref[pl.ds(mh + pq2, me), :] = yb[1, pl.ds(pq2, me), :].astype(
            jnp.float32
        )

        g3.wait()
        o1 = (1 - keep1) * mq
        out_ref[pl.ds(o1, mq), :] = yb[0, pl.ds(o1, mq), :].astype(jnp.float32)

        g4.wait()
        o2 = (1 - keep2) * mq
        out_ref[pl.ds(mh + o2, mq), :] = yb[1, pl.ds(o2, mq), :].astype(
            jnp.float32
        )

    return pl.pallas_call(
        body,
        out_shape=jax.ShapeDtypeStruct((m_per, n), jnp.float32),
        in_specs=[
            pl.BlockSpec(memory_space=pltpu.VMEM),
            pl.BlockSpec(memory_space=pltpu.VMEM),
        ],
        out_specs=pl.BlockSpec(memory_space=pltpu.VMEM),
        scratch_shapes=[
            pltpu.VMEM((2, mq, k), jnp.bfloat16),
            pltpu.VMEM((2, mq, k), jnp.bfloat16),
            pltpu.VMEM((2, me, k), jnp.bfloat16),
            pltpu.VMEM((2, me, k), jnp.bfloat16),
            pltpu.VMEM((2, mh, n), jnp.bfloat16),
            pltpu.SemaphoreType.DMA((8,)),
            pltpu.SemaphoreType.DMA((8,)),
        ],
        compiler_params=pltpu.CompilerParams(collective_id=0),
    )(t, W)


# device time: 53011 ns/iter; 2.9811x vs baseline; 1.0205x over previous
import jax
import jax.numpy as jnp
from jax import lax
from jax.experimental import pallas as pl
from jax.experimental.pallas import tpu as pltpu

N_DEV = 4


def kernel(t, W):
    m_per, k = t.shape
    _, n = W.shape
    mh = m_per // 2
    mq = m_per // 4
    me = m_per // 8

    def body(t_ref, w_ref, out_ref, c1s, rs1r, rs2s, rs2r, yb, wb, sems_s, sems_r):
        my = lax.axis_index("i")
        p_a = my ^ 1
        p_b = 3 - my
        a_bit = my & 1
        b_bit = my // 2
        keep1 = a_bit ^ b_bit
        keep2 = b_bit
        q1 = b_bit
        q2 = a_bit
        row1 = keep1 * mq + q1 * me
        row2 = mh + keep2 * mq + q2 * me
        l2 = keep2 * mq + q2 * me

        barrier_sem = pltpu.get_barrier_semaphore()
        for nbr in (p_a, p_b):
            pl.semaphore_signal(
                barrier_sem, inc=1,
                device_id=(nbr,), device_id_type=pl.DeviceIdType.MESH,
            )
        pl.semaphore_wait(barrier_sem, 2)

        def xchg(sem_idx, src, dst, target):
            return pltpu.make_async_remote_copy(
                src_ref=src, dst_ref=dst,
                send_sem=sems_s.at[sem_idx], recv_sem=sems_r.at[sem_idx],
                device_id=(target,), device_id_type=pl.DeviceIdType.MESH,
            )

        c1s[0, :, :] = t_ref[pl.ds((1 - keep1) * mq, mq), :].astype(jnp.bfloat16)
        r1 = xchg(0, c1s.at[0], rs1r.at[0], p_a)
        r1.start()
        c1s[1, :, :] = t_ref[pl.ds(mh + (1 - keep2) * mq, mq), :].astype(
            jnp.bfloat16
        )
        r2 = xchg(1, c1s.at[1], rs1r.at[1], p_b)
        r2.start()

        wb[:, :] = w_ref[:, :].astype(jnp.bfloat16)

        r1.wait()
        rs2s[0, :, :] = (
            rs1r[0, pl.ds((1 - q1) * me, me), :].astype(jnp.float32)
            + t_ref[pl.ds(keep1 * mq + (1 - q1) * me, me), :]
        ).astype(jnp.bfloat16)
        r3 = xchg(2, rs2s.at[0], rs2r.at[0], p_b)
        r3.start()

        r2.wait()
        rs2s[1, :, :] = (
            rs1r[1, pl.ds((1 - q2) * me, me), :].astype(jnp.float32)
            + t_ref[pl.ds(mh + keep2 * mq + (1 - q2) * me, me), :]
        ).astype(jnp.bfloat16)
        r4 = xchg(3, rs2s.at[1], rs2r.at[1], p_a)
        r4.start()

        r3.wait()
        s1 = (
            rs2r[0, :, :].astype(jnp.float32)
            + rs1r[0, pl.ds(q1 * me, me), :].astype(jnp.float32)
            + t_ref[pl.ds(row1, me), :]
        )
        y1 = lax.dot_general(
            s1.astype(jnp.bfloat16), wb[:, :],
            dimension_numbers=(((1,), (0,)), ((), ())),
            preferred_element_type=jnp.float32,
        )
        out_ref[pl.ds(row1, me), :] = y1
        yb[0, pl.ds(row1, me), :] = y1.astype(jnp.bfloat16)
        g1 = xchg(4, yb.at[0, pl.ds(row1, me), :],
                  yb.at[0, pl.ds(row1, me), :], p_b)
        g1.start()

        r4.wait()
        s2 = (
            rs2r[1, :, :].astype(jnp.float32)
            + rs1r[1, pl.ds(q2 * me, me), :].astype(jnp.float32)
            + t_ref[pl.ds(row2, me), :]
        )
        y2 = lax.dot_general(
            s2.astype(jnp.bfloat16), wb[:, :],
            dimension_numbers=(((1,), (0,)), ((), ())),
            preferred_element_type=jnp.float32,
        )
        out_ref[pl.ds(row2, me), :] = y2
        yb[1, pl.ds(l2, me), :] = y2.astype(jnp.bfloat16)
        g2 = xchg(5, yb.at[1, pl.ds(l2, me), :],
                  yb.at[1, pl.ds(l2, me), :], p_a)
        g2.start()

        g1.wait()
        g3 = xchg(6, yb.at[0, pl.ds(keep1 * mq, mq), :],
                  yb.at[0, pl.ds(keep1 * mq, mq), :], p_a)
        g3.start()
        pq1 = keep1 * mq + (1 - q1) * me
        out_ref[pl.ds(pq1, me), :] = yb[0, pl.ds(pq1, me), :].astype(jnp.float32)

        g2.wait()
        g4 = xchg(7, yb.at[1, pl.ds(keep2 * mq, mq), :],
                  yb.at[1, pl.ds(keep2 * mq, mq), :], p_b)
        g4.start()
        pq2 = keep2 * mq + (1 - q2) * me
        out_ref[pl.ds(mh + pq2, me), :] = yb[1, pl.ds(pq2, me), :].astype(
            jnp.float32
        )

        g3.wait()
        o1 = (1 - keep1) * mq
        out_ref[pl.ds(o1, mq), :] = yb[0, pl.ds(o1, mq), :].astype(jnp.float32)

        g4.wait()
        o2 = (1 - keep2) * mq
        out_ref[pl.ds(mh + o2, mq), :] = yb[1, pl.ds(o2, mq), :].astype(
            jnp.float32
        )

    return pl.pallas_call(
        body,
        out_shape=jax.ShapeDtypeStruct((m_per, n), jnp.float32),
        in_specs=[
            pl.BlockSpec(memory_space=pltpu.VMEM),
            pl.BlockSpec(memory_space=pltpu.VMEM),
        ],
        out_specs=pl.BlockSpec(memory_space=pltpu.VMEM),
        scratch_shapes=[
            pltpu.VMEM((2, mq, k), jnp.bfloat16),
            pltpu.VMEM((2, mq, k), jnp.bfloat16),
            pltpu.VMEM((2, me, k), jnp.bfloat16),
            pltpu.VMEM((2, me, k), jnp.bfloat16),
            pltpu.VMEM((2, mh, n), jnp.bfloat16),
            pltpu.VMEM((k, n), jnp.bfloat16),
            pltpu.SemaphoreType.DMA((8,)),
            pltpu.SemaphoreType.DMA((8,)),
        ],
        compiler_params=pltpu.CompilerParams(collective_id=0),
    )(t, W)
